# baseline (speedup 1.0000x reference)
import functools

import jax
import jax.numpy as jnp
from jax import lax
from jax.experimental import pallas as pl
from jax.experimental.pallas import tpu as pltpu
from jax.experimental.pallas import tpu_sc as plsc

B = 16384
D = 64
N_SPARSE = 8192
NC = 2
NS = 16
NW = NC * NS
ROWS_PER_W = B // NW
CHUNK = 128
NCHUNK = ROWS_PER_W // CHUNK
NVEC = N_SPARSE // 16


def _sc_gather_and_resolve(fixed_features, sf_index, sf_value, fixed_table,
                           sparse_table):
    mesh = plsc.VectorSubcoreMesh(core_axis_name="c", subcore_axis_name="s",
                                  num_cores=NC, num_subcores=NS)

    @functools.partial(
        pl.kernel,
        out_type=(
            jax.ShapeDtypeStruct((B, D), jnp.float32),
            jax.ShapeDtypeStruct((B, D), jnp.float32),
            jax.ShapeDtypeStruct((B,), jnp.int32),
        ),
        mesh=mesh,
        scratch_types=[
            pltpu.VMEM((NCHUNK, CHUNK), jnp.int32),
            pltpu.VMEM((N_SPARSE,), jnp.int32),
            pltpu.VMEM((N_SPARSE,), jnp.int32),
            pltpu.VMEM((NCHUNK, CHUNK), jnp.int32),
            pltpu.VMEM((NCHUNK, CHUNK), jnp.int32),
            pltpu.VMEM((32,), jnp.int32),
            pltpu.VMEM((ROWS_PER_W, D), jnp.float32),
            pltpu.VMEM((ROWS_PER_W, D), jnp.float32),
            pltpu.SemaphoreType.DMA,
        ],
        compiler_params=pltpu.CompilerParams(needs_layout_passes=False,
                                             use_tc_tiling_on_sc=False),
    )
    def k(ff_hbm, sfi_hbm, sfv_hbm, ftab_hbm, stab_hbm,
          fe_out, sr_out, sv_out,
          ffeat, sfi, sfv, sval, svc, scr, fe_rows, srows, sem):
        wid = lax.axis_index("c") * NS + lax.axis_index("s")
        base = wid * ROWS_PER_W

        for j in range(NCHUNK):
            pltpu.sync_copy(ff_hbm.at[pl.ds(base + CHUNK * j, CHUNK)],
                            ffeat.at[j])
        fcopies = [
            pltpu.async_copy(ftab_hbm.at[ffeat.at[j]],
                             fe_rows.at[pl.ds(CHUNK * j, CHUNK)], sem)
            for j in range(NCHUNK)
        ]

        pltpu.sync_copy(sfi_hbm, sfi)
        pltpu.sync_copy(sfv_hbm, sfv)

        neg1 = jnp.full((16,), -1, jnp.int32)
        for j in range(NCHUNK):
            for g in range(CHUNK // 16):
                sval[j, pl.ds(16 * g, 16)] = neg1
        scr[pl.ds(16, 16)] = neg1

        iota = lax.iota(jnp.int32, 16)
        iota1 = iota + 1

        def scan_body(v, carry):
            b16 = sfi[pl.ds(16 * v, 16)]

            @pl.when(jnp.any((b16 >> 9) == wid))
            def _():
                key = (b16 << 13) | (iota + 16 * v)
                key_s = jnp.sort(key)
                scr[pl.ds(0, 16)] = key_s
                nxt = plsc.load_gather(scr, [iota1])
                b_s = key_s >> 13
                i_s = key_s & (N_SPARSE - 1)
                winner = (b_s != (nxt >> 13)) & ((b_s >> 9) == wid)
                sval_s = plsc.load_gather(sfv, [i_s])
                bl = b_s & (ROWS_PER_W - 1)
                plsc.store_scatter(sval, [bl >> 7, bl & (CHUNK - 1)],
                                   sval_s, mask=winner)

            return carry

        lax.fori_loop(0, NVEC, scan_body, 0)

        for j in range(NCHUNK):
            for g in range(CHUNK // 16):
                x = sval[j, pl.ds(16 * g, 16)]
                svc[j, pl.ds(16 * g, 16)] = jnp.maximum(x, 0)

        scopies = [
            pltpu.async_copy(stab_hbm.at[svc.at[j]],
                             srows.at[pl.ds(CHUNK * j, CHUNK)], sem)
            for j in range(NCHUNK)
        ]
        for c in fcopies + scopies:
            c.wait()

        pltpu.sync_copy(fe_rows, fe_out.at[pl.ds(base, ROWS_PER_W)])
        pltpu.sync_copy(srows, sr_out.at[pl.ds(base, ROWS_PER_W)])
        for j in range(NCHUNK):
            pltpu.sync_copy(sval.at[j],
                            sv_out.at[pl.ds(base + CHUNK * j, CHUNK)])

    return k(fixed_features, sf_index, sf_value, fixed_table, sparse_table)


BLK = 2048


def _mm_body(fe_ref, se_ref, sv_ref, w1_ref, w2_ref, b_ref, o_ref):
    mask = (sv_ref[...] >= 0).astype(jnp.float32)
    se = se_ref[...] * mask
    o_ref[...] = (
        jnp.dot(fe_ref[...], w1_ref[...], preferred_element_type=jnp.float32)
        + jnp.dot(se, w2_ref[...], preferred_element_type=jnp.float32)
        + b_ref[...]
    )


def _tc_matmul(fe, se, sv, W, b):
    grid = (B // BLK,)
    return pl.pallas_call(
        _mm_body,
        grid=grid,
        in_specs=[
            pl.BlockSpec((BLK, D), lambda i: (i, 0)),
            pl.BlockSpec((BLK, D), lambda i: (i, 0)),
            pl.BlockSpec((BLK, 1), lambda i: (i, 0)),
            pl.BlockSpec((D, D), lambda i: (0, 0)),
            pl.BlockSpec((D, D), lambda i: (0, 0)),
            pl.BlockSpec((1, D), lambda i: (0, 0)),
        ],
        out_specs=pl.BlockSpec((BLK, D), lambda i: (i, 0)),
        out_shape=jax.ShapeDtypeStruct((B, D), jnp.float32),
    )(fe, se, sv, W[:D], W[D:], b.reshape(1, D))


def kernel(fixed_features, sf_index, sf_value, fixed_table, sparse_table, W, b):
    fe, se, sv = _sc_gather_and_resolve(
        fixed_features.astype(jnp.int32),
        sf_index.astype(jnp.int32),
        sf_value.astype(jnp.int32),
        fixed_table, sparse_table)
    return _tc_matmul(fe, se, sv.reshape(B, 1), W, b)


# speedup vs baseline: 1.0115x; 1.0115x over previous
import functools

import jax
import jax.numpy as jnp
from jax import lax
from jax.experimental import pallas as pl
from jax.experimental.pallas import tpu as pltpu
from jax.experimental.pallas import tpu_sc as plsc

B = 16384
D = 64
N_SPARSE = 8192
NC = 2
NS = 16
NW = NC * NS
ROWS_PER_W = B // NW
CHUNK = 128
NCHUNK = ROWS_PER_W // CHUNK
NVEC = N_SPARSE // 16


def _sc_gather_and_resolve(fixed_features, sf_index, sf_value, fixed_table,
                           sparse_table):
    mesh = plsc.VectorSubcoreMesh(core_axis_name="c", subcore_axis_name="s",
                                  num_cores=NC, num_subcores=NS)

    @functools.partial(
        pl.kernel,
        out_type=(
            jax.ShapeDtypeStruct((B, D), jnp.float32),
            jax.ShapeDtypeStruct((B, D), jnp.float32),
            jax.ShapeDtypeStruct((B,), jnp.int32),
        ),
        mesh=mesh,
        scratch_types=[
            pltpu.VMEM((NCHUNK, CHUNK), jnp.int32),
            pltpu.VMEM((N_SPARSE,), jnp.int32),
            pltpu.VMEM((N_SPARSE,), jnp.int32),
            pltpu.VMEM((NCHUNK, CHUNK), jnp.int32),
            pltpu.VMEM((NCHUNK, CHUNK), jnp.int32),
            pltpu.VMEM((32,), jnp.int32),
            pltpu.VMEM((ROWS_PER_W, D), jnp.float32),
            pltpu.VMEM((ROWS_PER_W, D), jnp.float32),
            pltpu.SemaphoreType.DMA,
        ],
        compiler_params=pltpu.CompilerParams(needs_layout_passes=False,
                                             use_tc_tiling_on_sc=False),
    )
    def k(ff_hbm, sfi_hbm, sfv_hbm, ftab_hbm, stab_hbm,
          fe_out, sr_out, sv_out,
          ffeat, sfi, sfv, sval, svc, scr, fe_rows, srows, sem):
        wid = lax.axis_index("c") * NS + lax.axis_index("s")
        base = wid * ROWS_PER_W

        for j in range(NCHUNK):
            pltpu.sync_copy(ff_hbm.at[pl.ds(base + CHUNK * j, CHUNK)],
                            ffeat.at[j])
        fcopies = [
            pltpu.async_copy(ftab_hbm.at[ffeat.at[j]],
                             fe_rows.at[pl.ds(CHUNK * j, CHUNK)], sem)
            for j in range(NCHUNK)
        ]

        pltpu.sync_copy(sfi_hbm, sfi)
        pltpu.sync_copy(sfv_hbm, sfv)

        neg1 = jnp.full((16,), -1, jnp.int32)
        for j in range(NCHUNK):
            for g in range(CHUNK // 16):
                sval[j, pl.ds(16 * g, 16)] = neg1
        scr[pl.ds(16, 16)] = neg1

        iota = lax.iota(jnp.int32, 16)
        iota1 = iota + 1

        UNROLL = 8

        def scan_body(u, carry):
            for s in range(UNROLL):
                v = u * UNROLL + s
                b16 = sfi[pl.ds(16 * v, 16)]
                key = (b16 << 13) | (iota + 16 * v)
                key_s = jnp.sort(key)
                scr[pl.ds(0, 16)] = key_s
                nxt = plsc.load_gather(scr, [iota1])
                b_s = key_s >> 13
                i_s = key_s & (N_SPARSE - 1)
                winner = (b_s != (nxt >> 13)) & ((b_s >> 9) == wid)
                sval_s = plsc.load_gather(sfv, [i_s])
                bl = b_s & (ROWS_PER_W - 1)
                plsc.store_scatter(sval, [bl >> 7, bl & (CHUNK - 1)],
                                   sval_s, mask=winner)
            return carry

        lax.fori_loop(0, NVEC // UNROLL, scan_body, 0)

        for j in range(NCHUNK):
            for g in range(CHUNK // 16):
                x = sval[j, pl.ds(16 * g, 16)]
                svc[j, pl.ds(16 * g, 16)] = jnp.maximum(x, 0)

        scopies = [
            pltpu.async_copy(stab_hbm.at[svc.at[j]],
                             srows.at[pl.ds(CHUNK * j, CHUNK)], sem)
            for j in range(NCHUNK)
        ]
        for c in fcopies + scopies:
            c.wait()

        pltpu.sync_copy(fe_rows, fe_out.at[pl.ds(base, ROWS_PER_W)])
        pltpu.sync_copy(srows, sr_out.at[pl.ds(base, ROWS_PER_W)])
        for j in range(NCHUNK):
            pltpu.sync_copy(sval.at[j],
                            sv_out.at[pl.ds(base + CHUNK * j, CHUNK)])

    return k(fixed_features, sf_index, sf_value, fixed_table, sparse_table)


BLK = 2048


def _mm_body(fe_ref, se_ref, sv_ref, w1_ref, w2_ref, b_ref, o_ref):
    mask = (sv_ref[...] >= 0).astype(jnp.float32)
    se = se_ref[...] * mask
    o_ref[...] = (
        jnp.dot(fe_ref[...], w1_ref[...], preferred_element_type=jnp.float32)
        + jnp.dot(se, w2_ref[...], preferred_element_type=jnp.float32)
        + b_ref[...]
    )


def _tc_matmul(fe, se, sv, W, b):
    grid = (B // BLK,)
    return pl.pallas_call(
        _mm_body,
        grid=grid,
        in_specs=[
            pl.BlockSpec((BLK, D), lambda i: (i, 0)),
            pl.BlockSpec((BLK, D), lambda i: (i, 0)),
            pl.BlockSpec((BLK, 1), lambda i: (i, 0)),
            pl.BlockSpec((D, D), lambda i: (0, 0)),
            pl.BlockSpec((D, D), lambda i: (0, 0)),
            pl.BlockSpec((1, D), lambda i: (0, 0)),
        ],
        out_specs=pl.BlockSpec((BLK, D), lambda i: (i, 0)),
        out_shape=jax.ShapeDtypeStruct((B, D), jnp.float32),
    )(fe, se, sv, W[:D], W[D:], b.reshape(1, D))


def kernel(fixed_features, sf_index, sf_value, fixed_table, sparse_table, W, b):
    fe, se, sv = _sc_gather_and_resolve(
        fixed_features.astype(jnp.int32),
        sf_index.astype(jnp.int32),
        sf_value.astype(jnp.int32),
        fixed_table, sparse_table)
    return _tc_matmul(fe, se, sv.reshape(B, 1), W, b)
